# Initial kernel scaffold; baseline (speedup 1.0000x reference)
import jax
import jax.numpy as jnp
from jax.experimental import pallas as pl


def kernel(x, edge_index, W, att_src, att_dst, bias, gamma, beta):
    raise NotImplementedError("write your pallas kernel here")


# speedup vs baseline: 15.0328x; 15.0328x over previous
import functools

import jax
import jax.numpy as jnp
from jax import lax
from jax.experimental import pallas as pl
from jax.experimental.pallas import tpu as pltpu
from jax.experimental.pallas import tpu_sc as plsc

N = 10000
D = 128
E = 320000
ACC_W = 144
NC = 2
NS = 16
NW = NC * NS
EDGES_PER_TILE = E // NW
CHUNK = 80
NCHUNK = EDGES_PER_TILE // CHUNK
ROWS_PER_TILE = N // NS
ZROWS = 125
RBLK = 1000


def _dense_body(x_ref, w_ref, as_ref, ad_ref, h_ref, asr_ref, adr_ref,
                mxs_ref, mxd_ref):
    h = lax.dot_general(x_ref[...], w_ref[...], (((1,), (1,)), ((), ())),
                        preferred_element_type=jnp.float32)
    h_ref[...] = h
    a_s = jnp.sum(h * as_ref[...], axis=1, keepdims=True)
    a_d = jnp.sum(h * ad_ref[...], axis=1, keepdims=True)
    asr_ref[...] = a_s
    adr_ref[...] = a_d

    @pl.when(pl.program_id(0) == 0)
    def _init():
        mxs_ref[...] = jnp.full((1, 1), -jnp.inf, jnp.float32)
        mxd_ref[...] = jnp.full((1, 1), -jnp.inf, jnp.float32)

    mxs_ref[...] = jnp.maximum(mxs_ref[...], jnp.full((1, 1), jnp.max(a_s)))
    mxd_ref[...] = jnp.maximum(mxd_ref[...], jnp.full((1, 1), jnp.max(a_d)))


_dense = pl.pallas_call(
    _dense_body,
    grid=(N // RBLK,),
    in_specs=[
        pl.BlockSpec((RBLK, D), lambda i: (i, 0)),
        pl.BlockSpec((D, D), lambda i: (0, 0)),
        pl.BlockSpec((1, D), lambda i: (0, 0)),
        pl.BlockSpec((1, D), lambda i: (0, 0)),
    ],
    out_specs=[
        pl.BlockSpec((RBLK, D), lambda i: (i, 0)),
        pl.BlockSpec((RBLK, 1), lambda i: (i, 0)),
        pl.BlockSpec((RBLK, 1), lambda i: (i, 0)),
        pl.BlockSpec((1, 1), lambda i: (0, 0)),
        pl.BlockSpec((1, 1), lambda i: (0, 0)),
    ],
    out_shape=[
        jax.ShapeDtypeStruct((N, D), jnp.float32),
        jax.ShapeDtypeStruct((N, 1), jnp.float32),
        jax.ShapeDtypeStruct((N, 1), jnp.float32),
        jax.ShapeDtypeStruct((1, 1), jnp.float32),
        jax.ShapeDtypeStruct((1, 1), jnp.float32),
    ],
)


_mesh = plsc.VectorSubcoreMesh(core_axis_name="c", subcore_axis_name="s")


@functools.partial(
    pl.kernel,
    mesh=_mesh,
    compiler_params=pltpu.CompilerParams(
        use_tc_tiling_on_sc=False, needs_layout_passes=False),
    out_type=jax.ShapeDtypeStruct((NC, N, ACC_W), jnp.float32),
    scratch_types=[
        pltpu.VMEM((16,), jnp.float32),
        pltpu.VMEM((CHUNK,), jnp.int32),
        pltpu.VMEM((CHUNK,), jnp.int32),
        pltpu.VMEM((CHUNK,), jnp.float32),
        pltpu.VMEM((CHUNK,), jnp.float32),
        pltpu.VMEM((CHUNK, D), jnp.float32),
        pltpu.VMEM((CHUNK, ACC_W), jnp.float32),
        pltpu.VMEM((CHUNK + 16,), jnp.float32),
        pltpu.VMEM_SHARED((N,), jnp.float32),
        pltpu.VMEM_SHARED((N,), jnp.float32),
        pltpu.VMEM_SHARED((N, ACC_W), jnp.float32),
        pltpu.SemaphoreType.DMA,
        pltpu.SemaphoreType.DMA,
    ],
)
def _edge_kernel(src_hbm, dst_hbm, asrc_hbm, adst_hbm, h_hbm, c_hbm, out_hbm,
                 c_v, src_v, dst_v, asv_v, adv_v, rows_v, wrows_v, w_v,
                 asrc_sh, adst_sh, acc, sem, sem2):
    cid = lax.axis_index("c")
    sid = lax.axis_index("s")
    wid = sid * NC + cid

    pltpu.sync_copy(c_hbm, c_v)
    cvec = c_v[...]

    @pl.when(sid == 0)
    def _stage():
        pltpu.sync_copy(asrc_hbm, asrc_sh)
        pltpu.sync_copy(adst_hbm, adst_sh)

    zv = jnp.zeros((16,), jnp.float32)

    def zrow(r, carry):
        for g in range(ACC_W // 16):
            wrows_v[r, pl.ds(g * 16, 16)] = zv
        return carry

    lax.fori_loop(0, CHUNK, zrow, 0)
    rbase = sid * ROWS_PER_TILE
    for piece in range(ROWS_PER_TILE // CHUNK):
        pltpu.sync_copy(wrows_v, acc.at[pl.ds(rbase + piece * CHUNK, CHUNK)])
    ztail = ROWS_PER_TILE % CHUNK
    if ztail:
        pltpu.sync_copy(
            wrows_v.at[pl.ds(0, ztail)],
            acc.at[pl.ds(rbase + (ROWS_PER_TILE // CHUNK) * CHUNK, ztail)])
    plsc.subcore_barrier()

    lane0 = jnp.where(lax.iota(jnp.int32, 16) == 0, 1.0, 0.0)
    ebase = wid * EDGES_PER_TILE

    def chunk(ci, carry):
        base = ebase + ci * CHUNK
        pltpu.sync_copy(src_hbm.at[pl.ds(base, CHUNK)], src_v)
        pltpu.sync_copy(dst_hbm.at[pl.ds(base, CHUNK)], dst_v)
        hcp = pltpu.async_copy(h_hbm.at[src_v], rows_v, sem)
        acp = pltpu.async_copy(asrc_sh.at[src_v], asv_v, sem2)
        bcp = pltpu.async_copy(adst_sh.at[dst_v], adv_v, sem2)
        acp.wait()
        bcp.wait()
        for g in range(CHUNK // 16):
            e = asv_v[pl.ds(g * 16, 16)] + adv_v[pl.ds(g * 16, 16)]
            e = jnp.where(e >= 0.0, e, 0.2 * e) - cvec
            w_v[pl.ds(g * 16, 16)] = jnp.exp(e)
        hcp.wait()

        def edge(ei, c2):
            wvec = w_v[pl.ds(ei, 16)]
            w = jnp.full((16,), wvec[0], jnp.float32)
            for g in range(D // 16):
                wrows_v[ei, pl.ds(g * 16, 16)] = rows_v[ei, pl.ds(g * 16, 16)] * w
            wrows_v[ei, pl.ds(D, 16)] = w * lane0
            return c2

        lax.fori_loop(0, CHUNK, edge, 0)
        pltpu.sync_copy(wrows_v, acc.at[dst_v], add=True)
        return carry

    lax.fori_loop(0, NCHUNK, chunk, 0)
    plsc.subcore_barrier()

    pltpu.sync_copy(
        acc.at[pl.ds(rbase, ROWS_PER_TILE)],
        out_hbm.at[cid, pl.ds(rbase, ROWS_PER_TILE)])


def _finalize_body(part_ref, h_ref, asr_ref, adr_ref, c_ref, bias_ref,
                   out_ref, ssum_ref, ssq_ref):
    c = c_ref[0, 0]
    acc = part_ref[0] + part_ref[1]
    num = acc[:, :D]
    den = acc[:, D:D + 1]
    s = asr_ref[...] + adr_ref[...]
    e = jnp.where(s >= 0.0, s, 0.2 * s) - c
    wself = jnp.exp(e)
    num = num + wself * h_ref[...]
    den = den + wself
    o = num / (den + 1e-16) + bias_ref[...]
    out_ref[...] = o

    @pl.when(pl.program_id(0) == 0)
    def _init():
        ssum_ref[...] = jnp.zeros((1, D), jnp.float32)
        ssq_ref[...] = jnp.zeros((1, D), jnp.float32)

    ssum_ref[...] += jnp.sum(o, axis=0, keepdims=True)
    ssq_ref[...] += jnp.sum(o * o, axis=0, keepdims=True)


_finalize = pl.pallas_call(
    _finalize_body,
    grid=(N // RBLK,),
    in_specs=[
        pl.BlockSpec((NC, RBLK, ACC_W), lambda i: (0, i, 0)),
        pl.BlockSpec((RBLK, D), lambda i: (i, 0)),
        pl.BlockSpec((RBLK, 1), lambda i: (i, 0)),
        pl.BlockSpec((RBLK, 1), lambda i: (i, 0)),
        pl.BlockSpec(memory_space=pltpu.SMEM),
        pl.BlockSpec((1, D), lambda i: (0, 0)),
    ],
    out_specs=[
        pl.BlockSpec((RBLK, D), lambda i: (i, 0)),
        pl.BlockSpec((1, D), lambda i: (0, 0)),
        pl.BlockSpec((1, D), lambda i: (0, 0)),
    ],
    out_shape=[
        jax.ShapeDtypeStruct((N, D), jnp.float32),
        jax.ShapeDtypeStruct((1, D), jnp.float32),
        jax.ShapeDtypeStruct((1, D), jnp.float32),
    ],
)


def _norm_body(xp_ref, ssum_ref, ssq_ref, g_ref, b_ref, o_ref):
    mu = ssum_ref[...] * (1.0 / N)
    var = ssq_ref[...] * (1.0 / N) - mu * mu
    scale = g_ref[...] * lax.rsqrt(var + 1e-5)
    o_ref[...] = (xp_ref[...] - mu) * scale + b_ref[...]


_normalize = pl.pallas_call(
    _norm_body,
    grid=(N // RBLK,),
    in_specs=[
        pl.BlockSpec((RBLK, D), lambda i: (i, 0)),
        pl.BlockSpec((1, D), lambda i: (0, 0)),
        pl.BlockSpec((1, D), lambda i: (0, 0)),
        pl.BlockSpec((1, D), lambda i: (0, 0)),
        pl.BlockSpec((1, D), lambda i: (0, 0)),
    ],
    out_specs=pl.BlockSpec((RBLK, D), lambda i: (i, 0)),
    out_shape=jax.ShapeDtypeStruct((N, D), jnp.float32),
)


def kernel(x, edge_index, W, att_src, att_dst, bias, gamma, beta):
    src = edge_index[0].astype(jnp.int32)
    dst = edge_index[1].astype(jnp.int32)
    h, asr, adr, mxs, mxd = _dense(
        x, W, att_src.reshape(1, D), att_dst.reshape(1, D))
    s = mxs[0, 0] + mxd[0, 0]
    c = jnp.where(s >= 0.0, s, 0.2 * s)
    cvec = jnp.full((16,), c, jnp.float32)
    part = _edge_kernel(src, dst, asr.reshape(N), adr.reshape(N), h, cvec)
    out_pre, ssum, ssq = _finalize(
        part, h, asr, adr, c.reshape(1, 1), bias.reshape(1, D))
    return _normalize(out_pre, ssum, ssq, gamma.reshape(1, D),
                      beta.reshape(1, D))
